# Initial kernel scaffold; baseline (speedup 1.0000x reference)
import jax
import jax.numpy as jnp
from jax.experimental import pallas as pl


def kernel(node_features, node_types, edge_index_c, edge_attr_c, edge_index_i, edge_attr_i, edge_index_p, edge_attr_p, type_emb, gates, etype_att, W0, a_src0, a_dst0, We0, a_edge0, b0, W1, a_src1, a_dst1, We1, a_edge1, b1):
    raise NotImplementedError("write your pallas kernel here")


# speedup vs baseline: 1.1989x; 1.1989x over previous
import functools

import jax
import jax.numpy as jnp
from jax.experimental import pallas as pl

N = 50000
E = 800000
D_FEAT = 9
HID = 64
HEADS = 4
C0 = HID // HEADS

_BN = 2000


def _prep_body(nf_ref, nt_ref, w_ref, m_ref, as_ref, ad_ref, xw_ref, als_ref, ald_ref):
    x = jnp.dot(nf_ref[...], w_ref[...], preferred_element_type=jnp.float32)
    t = nt_ref[...]
    sel = jnp.where(t == 0, m_ref[0:1, :], m_ref[1:2, :])
    xw = x + sel
    xw_ref[...] = xw
    als_ref[...] = jnp.dot(xw, as_ref[...], preferred_element_type=jnp.float32)
    ald_ref[...] = jnp.dot(xw, ad_ref[...], preferred_element_type=jnp.float32)


def _prep(nf, nt2d, w, m, a_s, a_d, heads):
    grid = (N // _BN,)
    kin = nf.shape[1]
    return pl.pallas_call(
        _prep_body,
        grid=grid,
        in_specs=[
            pl.BlockSpec((_BN, kin), lambda i: (i, 0)),
            pl.BlockSpec((_BN, 1), lambda i: (i, 0)),
            pl.BlockSpec((kin, HID), lambda i: (0, 0)),
            pl.BlockSpec((2, HID), lambda i: (0, 0)),
            pl.BlockSpec((HID, heads), lambda i: (0, 0)),
            pl.BlockSpec((HID, heads), lambda i: (0, 0)),
        ],
        out_specs=[
            pl.BlockSpec((_BN, HID), lambda i: (i, 0)),
            pl.BlockSpec((_BN, heads), lambda i: (i, 0)),
            pl.BlockSpec((_BN, heads), lambda i: (i, 0)),
        ],
        out_shape=[
            jax.ShapeDtypeStruct((N, HID), jnp.float32),
            jax.ShapeDtypeStruct((N, heads), jnp.float32),
            jax.ShapeDtypeStruct((N, heads), jnp.float32),
        ],
    )(nf, nt2d, w, m, a_s, a_d)


def _agg(src, dst, al_s, al_d, al_e, xw, heads):
    ch = HID // heads
    alpha = al_s[src] + al_d[dst] + al_e
    alpha = jnp.where(alpha > 0, alpha, 0.2 * alpha)
    ex = jnp.exp(alpha)
    denom = jax.ops.segment_sum(ex, dst, num_segments=N)
    msg = xw[src].reshape(E, heads, ch) * ex[:, :, None]
    num = jax.ops.segment_sum(msg, dst, num_segments=N)
    return num / (denom[:, :, None] + 1e-16)


def kernel(node_features, node_types, edge_index_c, edge_attr_c, edge_index_i,
           edge_attr_i, edge_index_p, edge_attr_p, type_emb, gates, etype_att,
           W0, a_src0, a_dst0, We0, a_edge0, b0, W1, a_src1, a_dst1, We1,
           a_edge1, b1):
    gate = jax.nn.sigmoid(gates)
    ew = jax.nn.softmax(etype_att)
    nt2d = node_types.astype(jnp.int32).reshape(N, 1)
    eis = [edge_index_c, edge_index_i, edge_index_p]
    eas = [edge_attr_c, edge_attr_i, edge_attr_p]

    total = jnp.zeros((N, HID), jnp.float32)
    for i in range(3):
        src = eis[i][0]
        dst = eis[i][1]
        attr = eas[i]

        W = W0[i]
        M0 = type_emb @ W[D_FEAT:]
        As0 = jnp.zeros((HID, HEADS), jnp.float32)
        Ad0 = jnp.zeros((HID, HEADS), jnp.float32)
        for h in range(HEADS):
            As0 = As0.at[h * C0:(h + 1) * C0, h].set(a_src0[i, h])
            Ad0 = Ad0.at[h * C0:(h + 1) * C0, h].set(a_dst0[i, h])
        B0 = jnp.einsum('dhc,hc->dh', We0[i].reshape(-1, HEADS, C0),
                        a_edge0[i]) * gate[i]

        xw, al_s, al_d = _prep(node_features, nt2d, W[:D_FEAT], M0, As0, Ad0, HEADS)
        al_e = attr @ B0

        out0 = _agg(src, dst, al_s, al_d, al_e, xw, HEADS)
        h0 = out0.reshape(N, HID) + b0[i]
        h0 = jnp.where(h0 > 0, h0, jnp.expm1(h0))

        M1 = jnp.zeros((2, HID), jnp.float32)
        As1 = a_src1[i].reshape(HID, 1)
        Ad1 = a_dst1[i].reshape(HID, 1)
        B1 = (We1[i] @ a_edge1[i, 0]) * gate[i]

        hw, al_s1, al_d1 = _prep(h0, nt2d, W1[i], M1, As1, Ad1, 1)
        al_e1 = (attr @ B1).reshape(E, 1)

        out1 = _agg(src, dst, al_s1, al_d1, al_e1, hw, 1)
        total = total + (out1.reshape(N, HID) + b1[i]) * ew[i]

    return total


# speedup vs baseline: 47.8519x; 39.9134x over previous
import jax
import jax.numpy as jnp
from jax import lax
from jax.experimental import pallas as pl
from jax.experimental.pallas import tpu as pltpu
from jax.experimental.pallas import tpu_sc as plsc

N = 50000
E = 800000
D_FEAT = 9
HID = 64
HEADS = 4
C0 = HID // HEADS

N_PAD = 50048
E_PAD = 802816
_NW = 32
_K = 512
_EPW = E_PAD // _NW
_NCH = _EPW // _K
_NPW = N_PAD // 16

_BN = 2000


def _prep_body(nf_ref, nt_ref, w_ref, m_ref, as_ref, ad_ref, xw_ref, als_ref, ald_ref):
    x = jnp.dot(nf_ref[...], w_ref[...], preferred_element_type=jnp.float32)
    t = nt_ref[...]
    sel = jnp.where(t == 0, m_ref[0:1, :], m_ref[1:2, :])
    xw = x + sel
    xw_ref[...] = xw
    als_ref[...] = jnp.dot(xw, as_ref[...], preferred_element_type=jnp.float32)
    ald_ref[...] = jnp.dot(xw, ad_ref[...], preferred_element_type=jnp.float32)


def _prep(nf, nt2d, w, m, a_s, a_d, heads):
    kin = nf.shape[1]
    return pl.pallas_call(
        _prep_body,
        grid=(N // _BN,),
        in_specs=[
            pl.BlockSpec((_BN, kin), lambda i: (i, 0)),
            pl.BlockSpec((_BN, 1), lambda i: (i, 0)),
            pl.BlockSpec((kin, HID), lambda i: (0, 0)),
            pl.BlockSpec((2, HID), lambda i: (0, 0)),
            pl.BlockSpec((HID, heads), lambda i: (0, 0)),
            pl.BlockSpec((HID, heads), lambda i: (0, 0)),
        ],
        out_specs=[
            pl.BlockSpec((_BN, HID), lambda i: (i, 0)),
            pl.BlockSpec((_BN, heads), lambda i: (i, 0)),
            pl.BlockSpec((_BN, heads), lambda i: (i, 0)),
        ],
        out_shape=[
            jax.ShapeDtypeStruct((N, HID), jnp.float32),
            jax.ShapeDtypeStruct((N, heads), jnp.float32),
            jax.ShapeDtypeStruct((N, heads), jnp.float32),
        ],
    )(nf, nt2d, w, m, a_s, a_d)


def _sc_ex_body(src_hbm, dst_hbm, ale_hbm, als_hbm, ald_hbm, zden_hbm,
                ex_hbm, den_hbm,
                als_v, ald_v, src_v, dst_v, ale_v, ex_v, den_s):
    cid = lax.axis_index("c")
    sid = lax.axis_index("s")
    wid = cid * 16 + sid

    pltpu.sync_copy(als_hbm, als_v)
    pltpu.sync_copy(ald_hbm, ald_v)

    @pl.when(sid == 0)
    def _():
        pltpu.sync_copy(zden_hbm, den_s)

    plsc.subcore_barrier()

    base = wid * _EPW

    def chunk(cc, carry):
        off = pl.multiple_of(base + cc * _K, _K)
        pltpu.sync_copy(src_hbm.at[pl.ds(off, _K)], src_v)
        pltpu.sync_copy(dst_hbm.at[pl.ds(off, _K)], dst_v)
        pltpu.sync_copy(ale_hbm.at[pl.ds(off, _K)], ale_v)

        def exloop(j, c):
            sl = pl.ds(j * 16, 16)
            a = (plsc.load_gather(als_v, [src_v[sl]])
                 + plsc.load_gather(ald_v, [dst_v[sl]])
                 + ale_v[sl])
            a = jnp.maximum(a, 0.2 * a)
            ex_v[sl] = jnp.exp(a)
            return c

        lax.fori_loop(0, _K // 16, exloop, 0, unroll=4)

        pltpu.sync_copy(ex_v, ex_hbm.at[pl.ds(off, _K)])
        pltpu.sync_copy(ex_v, den_s.at[dst_v], add=True)
        return carry

    lax.fori_loop(0, _NCH, chunk, 0)

    plsc.subcore_barrier()

    rsl = pl.ds(sid * _NPW, _NPW)
    orow = pl.multiple_of(cid * N_PAD + sid * _NPW, 8)
    pltpu.sync_copy(den_s.at[rsl], den_hbm.at[pl.ds(orow, _NPW)])


_sc_ex = pl.kernel(
    _sc_ex_body,
    out_type=[
        jax.ShapeDtypeStruct((E_PAD,), jnp.float32),
        jax.ShapeDtypeStruct((2 * N_PAD,), jnp.float32),
    ],
    mesh=plsc.VectorSubcoreMesh(core_axis_name="c", subcore_axis_name="s"),
    compiler_params=pltpu.CompilerParams(needs_layout_passes=False,
                                         use_tc_tiling_on_sc=False),
    scratch_types=[
        pltpu.VMEM((N_PAD,), jnp.float32),
        pltpu.VMEM((N_PAD,), jnp.float32),
        pltpu.VMEM((_K,), jnp.int32),
        pltpu.VMEM((_K,), jnp.int32),
        pltpu.VMEM((_K,), jnp.float32),
        pltpu.VMEM((_K,), jnp.float32),
        pltpu.VMEM_SHARED((N_PAD,), jnp.float32),
    ],
)


_CB = 2 * C0


def _sc_pay_body(src_hbm, dst_hbm, ex0_hbm, ex1_hbm, xw_hbm, znum_hbm,
                 num_hbm,
                 src_v, dst_v, ex0_v, ex1_v, rows_v, num_s, sem):
    cid = lax.axis_index("c")
    sid = lax.axis_index("s")
    wid = cid * 16 + sid

    @pl.when(sid == 0)
    def _():
        pltpu.sync_copy(znum_hbm, num_s)

    plsc.subcore_barrier()

    base = wid * _EPW

    def chunk(cc, carry):
        off = pl.multiple_of(base + cc * _K, _K)
        pltpu.sync_copy(src_hbm.at[pl.ds(off, _K)], src_v)
        pltpu.sync_copy(dst_hbm.at[pl.ds(off, _K)], dst_v)
        pltpu.sync_copy(ex0_hbm.at[pl.ds(off, _K)], ex0_v)
        pltpu.sync_copy(ex1_hbm.at[pl.ds(off, _K)], ex1_v)
        pltpu.async_copy(xw_hbm.at[src_v], rows_v, sem).wait()

        def scaleloop(j, c):
            e0 = plsc.load_gather(ex0_v, [jnp.full((16,), j, jnp.int32)])
            e1 = plsc.load_gather(ex1_v, [jnp.full((16,), j, jnp.int32)])
            rows_v[j, 0:C0] = rows_v[j, 0:C0] * e0
            rows_v[j, C0:_CB] = rows_v[j, C0:_CB] * e1
            return c

        lax.fori_loop(0, _K, scaleloop, 0, unroll=8)

        pltpu.sync_copy(rows_v, num_s.at[dst_v], add=True)
        return carry

    lax.fori_loop(0, _NCH, chunk, 0)

    plsc.subcore_barrier()

    rsl = pl.ds(sid * _NPW, _NPW)
    orow = pl.multiple_of(cid * N_PAD + sid * _NPW, 8)
    pltpu.sync_copy(num_s.at[rsl], num_hbm.at[pl.ds(orow, _NPW)])


_sc_pay = pl.kernel(
    _sc_pay_body,
    out_type=jax.ShapeDtypeStruct((2 * N_PAD, _CB), jnp.float32),
    mesh=plsc.VectorSubcoreMesh(core_axis_name="c", subcore_axis_name="s"),
    compiler_params=pltpu.CompilerParams(needs_layout_passes=False,
                                         use_tc_tiling_on_sc=False),
    scratch_types=[
        pltpu.VMEM((_K,), jnp.int32),
        pltpu.VMEM((_K,), jnp.int32),
        pltpu.VMEM((_K,), jnp.float32),
        pltpu.VMEM((_K,), jnp.float32),
        pltpu.VMEM((_K, _CB), jnp.float32),
        pltpu.VMEM_SHARED((N_PAD, _CB), jnp.float32),
        pltpu.SemaphoreType.DMA,
    ],
)


def _sc_layer(src_pad, dst_pad, ale_rows, als_rows, ald_rows, xw_pad, znum,
              zden, ngroups):
    exs, dens = [], []
    for g in range(ngroups):
        ex, den = _sc_ex(src_pad, dst_pad, ale_rows[g], als_rows[g],
                         ald_rows[g], zden)
        exs.append(ex)
        dens.append(den[:N_PAD] + den[N_PAD:])
    halves = []
    for p in range(2):
        e0 = exs[(2 * p) % ngroups]
        e1 = exs[(2 * p + 1) % ngroups]
        num = _sc_pay(src_pad, dst_pad, e0, e1,
                      xw_pad[:, 2 * p * C0:(2 * p + 2) * C0], znum)
        halves.append(num[:N_PAD] + num[N_PAD:])
    return jnp.concatenate(halves, axis=1), dens


def _pad_nodes(a):
    return jnp.pad(a, ((0, N_PAD - N),) + ((0, 0),) * (a.ndim - 1))


def kernel(node_features, node_types, edge_index_c, edge_attr_c, edge_index_i,
           edge_attr_i, edge_index_p, edge_attr_p, type_emb, gates, etype_att,
           W0, a_src0, a_dst0, We0, a_edge0, b0, W1, a_src1, a_dst1, We1,
           a_edge1, b1):
    gate = jax.nn.sigmoid(gates)
    ew = jax.nn.softmax(etype_att)
    nt2d = node_types.astype(jnp.int32).reshape(N, 1)
    eis = [edge_index_c, edge_index_i, edge_index_p]
    eas = [edge_attr_c, edge_attr_i, edge_attr_p]
    znum = jnp.zeros((N_PAD, _CB), jnp.float32)
    zden = jnp.zeros((N_PAD,), jnp.float32)

    total = jnp.zeros((N, HID), jnp.float32)
    for i in range(3):
        src = eis[i][0].astype(jnp.int32)
        dst = eis[i][1].astype(jnp.int32)
        attr = eas[i]
        src_pad = jnp.concatenate([src, jnp.zeros((E_PAD - E,), jnp.int32)])
        dst_pad = jnp.concatenate([dst, jnp.zeros((E_PAD - E,), jnp.int32)])

        W = W0[i]
        M0 = type_emb @ W[D_FEAT:]
        As0 = jnp.zeros((HID, HEADS), jnp.float32)
        Ad0 = jnp.zeros((HID, HEADS), jnp.float32)
        for h in range(HEADS):
            As0 = As0.at[h * C0:(h + 1) * C0, h].set(a_src0[i, h])
            Ad0 = Ad0.at[h * C0:(h + 1) * C0, h].set(a_dst0[i, h])
        B0 = jnp.einsum('dhc,hc->dh', We0[i].reshape(-1, HEADS, C0),
                        a_edge0[i]) * gate[i]

        xw, al_s, al_d = _prep(node_features, nt2d, W[:D_FEAT], M0, As0, Ad0, HEADS)
        ale_t = jnp.pad((attr @ B0).T, ((0, 0), (0, E_PAD - E)),
                        constant_values=-1e30)
        als_t = _pad_nodes(al_s).T
        ald_t = _pad_nodes(al_d).T

        num0, dens0 = _sc_layer(src_pad, dst_pad,
                                [ale_t[h] for h in range(HEADS)],
                                [als_t[h] for h in range(HEADS)],
                                [ald_t[h] for h in range(HEADS)],
                                _pad_nodes(xw), znum, zden, HEADS)
        den0 = jnp.stack([d[:N] for d in dens0], axis=1)
        out0 = num0[:N].reshape(N, HEADS, C0) / (den0[:, :, None] + 1e-16)
        h0 = out0.reshape(N, HID) + b0[i]
        h0 = jnp.where(h0 > 0, h0, jnp.expm1(h0))

        M1 = jnp.zeros((2, HID), jnp.float32)
        As1 = a_src1[i].reshape(HID, 1)
        Ad1 = a_dst1[i].reshape(HID, 1)
        B1 = (We1[i] @ a_edge1[i, 0]) * gate[i]

        hw, al_s1, al_d1 = _prep(h0, nt2d, W1[i], M1, As1, Ad1, 1)
        ale1 = jnp.pad(attr @ B1, (0, E_PAD - E), constant_values=-1e30)
        als1 = _pad_nodes(al_s1[:, 0])
        ald1 = _pad_nodes(al_d1[:, 0])

        num1, dens1 = _sc_layer(src_pad, dst_pad, [ale1], [als1], [ald1],
                                _pad_nodes(hw), znum, zden, 1)
        out1 = num1[:N] / (dens1[0][:N, None] + 1e-16) + b1[i]
        total = total + out1 * ew[i]

    return total


# speedup vs baseline: 68.3908x; 1.4292x over previous
import jax
import jax.numpy as jnp
from jax import lax
from jax.experimental import pallas as pl
from jax.experimental.pallas import tpu as pltpu
from jax.experimental.pallas import tpu_sc as plsc

N = 50000
E = 800000
D_FEAT = 9
HID = 64
HEADS = 4
C0 = HID // HEADS

N_PAD = 50048
E_PAD = 811008
_NW = 32
_EPW = E_PAD // _NW
_KA = 1584
_NCHA = _EPW // _KA
_KB = 256
_NCHB = _EPW // _KB
_NPW = N_PAD // 16

_BN = 2000


def _prep_body(nf_ref, nt_ref, w_ref, m_ref, as_ref, ad_ref, xw_ref, als_ref, ald_ref):
    x = jnp.dot(nf_ref[...], w_ref[...], preferred_element_type=jnp.float32)
    t = nt_ref[...]
    sel = jnp.where(t == 0, m_ref[0:1, :], m_ref[1:2, :])
    xw = x + sel
    xw_ref[...] = xw
    als_ref[...] = jnp.dot(xw, as_ref[...], preferred_element_type=jnp.float32)
    ald_ref[...] = jnp.dot(xw, ad_ref[...], preferred_element_type=jnp.float32)


def _prep(nf, nt2d, w, m, a_s, a_d, heads):
    kin = nf.shape[1]
    return pl.pallas_call(
        _prep_body,
        grid=(N // _BN,),
        in_specs=[
            pl.BlockSpec((_BN, kin), lambda i: (i, 0)),
            pl.BlockSpec((_BN, 1), lambda i: (i, 0)),
            pl.BlockSpec((kin, HID), lambda i: (0, 0)),
            pl.BlockSpec((2, HID), lambda i: (0, 0)),
            pl.BlockSpec((HID, heads), lambda i: (0, 0)),
            pl.BlockSpec((HID, heads), lambda i: (0, 0)),
        ],
        out_specs=[
            pl.BlockSpec((_BN, HID), lambda i: (i, 0)),
            pl.BlockSpec((_BN, heads), lambda i: (i, 0)),
            pl.BlockSpec((_BN, heads), lambda i: (i, 0)),
        ],
        out_shape=[
            jax.ShapeDtypeStruct((N, HID), jnp.float32),
            jax.ShapeDtypeStruct((N, heads), jnp.float32),
            jax.ShapeDtypeStruct((N, heads), jnp.float32),
        ],
    )(nf, nt2d, w, m, a_s, a_d)


def _sc_ex_body(src_hbm, dst_hbm, ale_hbm, als_hbm, ald_hbm, zden_hbm,
                ex_hbm, den_hbm,
                als_v, ald_v, src_v0, dst_v0, ale_v0, src_v1, dst_v1, ale_v1,
                ex_v, den_s, sem0, sem1):
    cid = lax.axis_index("c")
    sid = lax.axis_index("s")
    wid = cid * 16 + sid
    base = wid * _EPW
    bufs = ((src_v0, dst_v0, ale_v0, sem0), (src_v1, dst_v1, ale_v1, sem1))

    def issue(b, cc):
        sv, dv, av, sem = bufs[b]
        off = pl.multiple_of(base + lax.rem(cc, _NCHA) * _KA, 8)
        pltpu.async_copy(src_hbm.at[pl.ds(off, _KA)], sv, sem)
        pltpu.async_copy(dst_hbm.at[pl.ds(off, _KA)], dv, sem)
        pltpu.async_copy(ale_hbm.at[pl.ds(off, _KA)], av, sem)

    def wait(b):
        sv, dv, av, sem = bufs[b]
        pltpu.make_async_copy(src_hbm.at[pl.ds(0, _KA)], sv, sem).wait()
        pltpu.make_async_copy(dst_hbm.at[pl.ds(0, _KA)], dv, sem).wait()
        pltpu.make_async_copy(ale_hbm.at[pl.ds(0, _KA)], av, sem).wait()

    def compute(b, cc):
        sv, dv, av, _ = bufs[b]
        off = pl.multiple_of(base + cc * _KA, 8)

        def exloop(j, c):
            sl = pl.ds(j * 16, 16)
            a = (plsc.load_gather(als_v, [sv[sl]])
                 + plsc.load_gather(ald_v, [dv[sl]])
                 + av[sl])
            a = jnp.maximum(a, 0.2 * a)
            ex_v[sl] = jnp.exp(a)
            return c

        lax.fori_loop(0, _KA // 16, exloop, 0, unroll=8)
        pltpu.sync_copy(ex_v, ex_hbm.at[pl.ds(off, _KA)])
        pltpu.sync_copy(ex_v, den_s.at[dv], add=True)

    pltpu.sync_copy(als_hbm, als_v)
    pltpu.sync_copy(ald_hbm, ald_v)

    @pl.when(sid == 0)
    def _():
        pltpu.sync_copy(zden_hbm, den_s)

    plsc.subcore_barrier()

    issue(0, 0)

    def dbl(m, c):
        c0 = 2 * m
        issue(1, c0 + 1)
        wait(0)
        compute(0, c0)
        issue(0, c0 + 2)
        wait(1)
        compute(1, c0 + 1)
        return c

    lax.fori_loop(0, _NCHA // 2, dbl, 0)
    wait(0)

    plsc.subcore_barrier()

    rsl = pl.ds(sid * _NPW, _NPW)
    orow = pl.multiple_of(cid * N_PAD + sid * _NPW, 8)
    pltpu.sync_copy(den_s.at[rsl], den_hbm.at[pl.ds(orow, _NPW)])


_sc_ex = pl.kernel(
    _sc_ex_body,
    out_type=[
        jax.ShapeDtypeStruct((E_PAD,), jnp.float32),
        jax.ShapeDtypeStruct((2 * N_PAD,), jnp.float32),
    ],
    mesh=plsc.VectorSubcoreMesh(core_axis_name="c", subcore_axis_name="s"),
    compiler_params=pltpu.CompilerParams(needs_layout_passes=False,
                                         use_tc_tiling_on_sc=False),
    scratch_types=[
        pltpu.VMEM((N_PAD,), jnp.float32),
        pltpu.VMEM((N_PAD,), jnp.float32),
        pltpu.VMEM((_KA,), jnp.int32),
        pltpu.VMEM((_KA,), jnp.int32),
        pltpu.VMEM((_KA,), jnp.float32),
        pltpu.VMEM((_KA,), jnp.int32),
        pltpu.VMEM((_KA,), jnp.int32),
        pltpu.VMEM((_KA,), jnp.float32),
        pltpu.VMEM((_KA,), jnp.float32),
        pltpu.VMEM_SHARED((N_PAD,), jnp.float32),
        pltpu.SemaphoreType.DMA,
        pltpu.SemaphoreType.DMA,
    ],
)


_CB = 2 * C0


def _sc_pay_body(src_hbm, dst_hbm, ex0_hbm, ex1_hbm, xw_hbm, znum_hbm,
                 num_hbm,
                 src_v0, dst_v0, e0_v0, e1_v0, rows_v0,
                 src_v1, dst_v1, e0_v1, e1_v1, rows_v1,
                 src_v2, dst_v2, e0_v2, e1_v2, rows_v2,
                 num_s, semL0, semL1, semL2, semG0, semG1, semG2):
    cid = lax.axis_index("c")
    sid = lax.axis_index("s")
    wid = cid * 16 + sid
    base = wid * _EPW
    bufs = ((src_v0, dst_v0, e0_v0, e1_v0, rows_v0, semL0, semG0),
            (src_v1, dst_v1, e0_v1, e1_v1, rows_v1, semL1, semG1),
            (src_v2, dst_v2, e0_v2, e1_v2, rows_v2, semL2, semG2))

    def issue_lin(b, cc):
        sv, dv, e0v, e1v, _, semL, _ = bufs[b]
        off = pl.multiple_of(base + lax.rem(cc, _NCHB) * _KB, 8)
        pltpu.async_copy(src_hbm.at[pl.ds(off, _KB)], sv, semL)
        pltpu.async_copy(dst_hbm.at[pl.ds(off, _KB)], dv, semL)
        pltpu.async_copy(ex0_hbm.at[pl.ds(off, _KB)], e0v, semL)
        pltpu.async_copy(ex1_hbm.at[pl.ds(off, _KB)], e1v, semL)

    def wait_lin(b):
        sv, dv, e0v, e1v, _, semL, _ = bufs[b]
        pltpu.make_async_copy(src_hbm.at[pl.ds(0, _KB)], sv, semL).wait()
        pltpu.make_async_copy(dst_hbm.at[pl.ds(0, _KB)], dv, semL).wait()
        pltpu.make_async_copy(ex0_hbm.at[pl.ds(0, _KB)], e0v, semL).wait()
        pltpu.make_async_copy(ex1_hbm.at[pl.ds(0, _KB)], e1v, semL).wait()

    def issue_gat(b):
        sv, _, _, _, rv, _, semG = bufs[b]
        pltpu.async_copy(xw_hbm.at[sv], rv, semG)

    def wait_gat(b):
        sv, _, _, _, rv, _, semG = bufs[b]
        pltpu.make_async_copy(xw_hbm.at[sv], rv, semG).wait()

    def compute(b):
        _, dv, e0v, e1v, rv, _, _ = bufs[b]

        def scaleloop(j, c):
            e0 = plsc.load_gather(e0v, [jnp.full((16,), j, jnp.int32)])
            e1 = plsc.load_gather(e1v, [jnp.full((16,), j, jnp.int32)])
            rv[j, 0:C0] = rv[j, 0:C0] * e0
            rv[j, C0:_CB] = rv[j, C0:_CB] * e1
            return c

        lax.fori_loop(0, _KB, scaleloop, 0, unroll=8)
        pltpu.sync_copy(rv, num_s.at[dv], add=True)

    @pl.when(sid == 0)
    def _():
        pltpu.sync_copy(znum_hbm, num_s)

    plsc.subcore_barrier()

    issue_lin(0, 0)
    issue_lin(1, 1)
    issue_lin(2, 2)
    wait_lin(0)
    issue_gat(0)

    def stage(ba, bb, c):
        wait_gat(ba)
        wait_lin(bb)
        issue_gat(bb)
        compute(ba)
        issue_lin(ba, c + 3)

    def trip(m, c):
        c0 = 3 * m
        stage(0, 1, c0)
        stage(1, 2, c0 + 1)
        stage(2, 0, c0 + 2)
        return c

    lax.fori_loop(0, _NCHB // 3, trip, 0)
    wait_gat(0)
    wait_lin(1)
    wait_lin(2)

    plsc.subcore_barrier()

    rsl = pl.ds(sid * _NPW, _NPW)
    orow = pl.multiple_of(cid * N_PAD + sid * _NPW, 8)
    pltpu.sync_copy(num_s.at[rsl], num_hbm.at[pl.ds(orow, _NPW)])


_sc_pay = pl.kernel(
    _sc_pay_body,
    out_type=jax.ShapeDtypeStruct((2 * N_PAD, _CB), jnp.float32),
    mesh=plsc.VectorSubcoreMesh(core_axis_name="c", subcore_axis_name="s"),
    compiler_params=pltpu.CompilerParams(needs_layout_passes=False,
                                         use_tc_tiling_on_sc=False),
    scratch_types=(
        [pltpu.VMEM((_KB,), jnp.int32),
         pltpu.VMEM((_KB,), jnp.int32),
         pltpu.VMEM((_KB,), jnp.float32),
         pltpu.VMEM((_KB,), jnp.float32),
         pltpu.VMEM((_KB, _CB), jnp.float32)] * 3
        + [pltpu.VMEM_SHARED((N_PAD, _CB), jnp.float32)]
        + [pltpu.SemaphoreType.DMA] * 6
    ),
)


def _sc_layer(src_pad, dst_pad, ale_rows, als_rows, ald_rows, xw_pad, znum,
              zden, ngroups):
    exs, dens = [], []
    for g in range(ngroups):
        ex, den = _sc_ex(src_pad, dst_pad, ale_rows[g], als_rows[g],
                         ald_rows[g], zden)
        exs.append(ex)
        dens.append(den[:N_PAD] + den[N_PAD:])
    halves = []
    for p in range(2):
        e0 = exs[(2 * p) % ngroups]
        e1 = exs[(2 * p + 1) % ngroups]
        num = _sc_pay(src_pad, dst_pad, e0, e1,
                      xw_pad[:, 2 * p * C0:(2 * p + 2) * C0], znum)
        halves.append(num[:N_PAD] + num[N_PAD:])
    return jnp.concatenate(halves, axis=1), dens


def _pad_nodes(a):
    return jnp.pad(a, ((0, N_PAD - N),) + ((0, 0),) * (a.ndim - 1))


def kernel(node_features, node_types, edge_index_c, edge_attr_c, edge_index_i,
           edge_attr_i, edge_index_p, edge_attr_p, type_emb, gates, etype_att,
           W0, a_src0, a_dst0, We0, a_edge0, b0, W1, a_src1, a_dst1, We1,
           a_edge1, b1):
    gate = jax.nn.sigmoid(gates)
    ew = jax.nn.softmax(etype_att)
    nt2d = node_types.astype(jnp.int32).reshape(N, 1)
    eis = [edge_index_c, edge_index_i, edge_index_p]
    eas = [edge_attr_c, edge_attr_i, edge_attr_p]
    znum = jnp.zeros((N_PAD, _CB), jnp.float32)
    zden = jnp.zeros((N_PAD,), jnp.float32)

    total = jnp.zeros((N, HID), jnp.float32)
    for i in range(3):
        src = eis[i][0].astype(jnp.int32)
        dst = eis[i][1].astype(jnp.int32)
        attr = eas[i]
        src_pad = jnp.concatenate([src, jnp.zeros((E_PAD - E,), jnp.int32)])
        dst_pad = jnp.concatenate([dst, jnp.zeros((E_PAD - E,), jnp.int32)])

        W = W0[i]
        M0 = type_emb @ W[D_FEAT:]
        As0 = jnp.zeros((HID, HEADS), jnp.float32)
        Ad0 = jnp.zeros((HID, HEADS), jnp.float32)
        for h in range(HEADS):
            As0 = As0.at[h * C0:(h + 1) * C0, h].set(a_src0[i, h])
            Ad0 = Ad0.at[h * C0:(h + 1) * C0, h].set(a_dst0[i, h])
        B0 = jnp.einsum('dhc,hc->dh', We0[i].reshape(-1, HEADS, C0),
                        a_edge0[i]) * gate[i]

        xw, al_s, al_d = _prep(node_features, nt2d, W[:D_FEAT], M0, As0, Ad0, HEADS)
        ale_t = jnp.pad((attr @ B0).T, ((0, 0), (0, E_PAD - E)),
                        constant_values=-1e30)
        als_t = _pad_nodes(al_s).T
        ald_t = _pad_nodes(al_d).T

        num0, dens0 = _sc_layer(src_pad, dst_pad,
                                [ale_t[h] for h in range(HEADS)],
                                [als_t[h] for h in range(HEADS)],
                                [ald_t[h] for h in range(HEADS)],
                                _pad_nodes(xw), znum, zden, HEADS)
        den0 = jnp.stack([d[:N] for d in dens0], axis=1)
        out0 = num0[:N].reshape(N, HEADS, C0) / (den0[:, :, None] + 1e-16)
        h0 = out0.reshape(N, HID) + b0[i]
        h0 = jnp.where(h0 > 0, h0, jnp.expm1(h0))

        M1 = jnp.zeros((2, HID), jnp.float32)
        As1 = a_src1[i].reshape(HID, 1)
        Ad1 = a_dst1[i].reshape(HID, 1)
        B1 = (We1[i] @ a_edge1[i, 0]) * gate[i]

        hw, al_s1, al_d1 = _prep(h0, nt2d, W1[i], M1, As1, Ad1, 1)
        ale1 = jnp.pad(attr @ B1, (0, E_PAD - E), constant_values=-1e30)
        als1 = _pad_nodes(al_s1[:, 0])
        ald1 = _pad_nodes(al_d1[:, 0])

        num1, dens1 = _sc_layer(src_pad, dst_pad, [ale1], [als1], [ald1],
                                _pad_nodes(hw), znum, zden, 1)
        out1 = num1[:N] / (dens1[0][:N, None] + 1e-16) + b1[i]
        total = total + out1 * ew[i]

    return total
